# baseline (speedup 1.0000x reference)
import functools

import jax
import jax.numpy as jnp
from jax import lax
from jax.experimental import pallas as pl
from jax.experimental.pallas import tpu as pltpu
from jax.experimental.pallas import tpu_sc as plsc

B = 32
N_SNPS = 50000
N_GENES = 5000
N_NODES = 160000
N_FILT = 8
EPS = 1e-5
ROW = 128

NC = 2
NS = 16
NW = NC * NS

CHUNK = 128
N_CHUNKS = N_NODES // CHUNK
CHUNKS_PER_TILE = -(-N_CHUNKS // NW)
DEPTH = 5
GROUPS = CHUNKS_PER_TILE // DEPTH
SEG_PAD = 5120
ZROWS = SEG_PAD // NS


def _prep_body(snp_ref, filt_ref, table_ref):
    fmean = jnp.sum(filt_ref[...], axis=1) * (1.0 / N_FILT)
    s_t = jnp.transpose(snp_ref[...])
    blk = s_t.shape[0]
    table_ref[...] = jnp.concatenate(
        [s_t * fmean[:, None], jnp.zeros((blk, ROW - B), jnp.float32)],
        axis=1)


_PREP_BLK = 2048
_PREP_GRID = -(-N_SNPS // _PREP_BLK)


def _build_table(snp, filters):
    return pl.pallas_call(
        _prep_body,
        grid=(_PREP_GRID,),
        in_specs=[
            pl.BlockSpec((B, _PREP_BLK), lambda i: (0, i)),
            pl.BlockSpec((_PREP_BLK, N_FILT), lambda i: (i, 0)),
        ],
        out_specs=pl.BlockSpec((_PREP_BLK, ROW), lambda i: (i, 0)),
        out_shape=jax.ShapeDtypeStruct((N_SNPS, ROW), jnp.float32),
    )(snp, filters)


def _bag_body(ids_hbm, segs_hbm, table_hbm, out_hbm,
              idx0, idx1, idx2, idx3, idx4, seg0, seg1, seg2, seg3, seg4,
              rows, zbuf, acc, sem0, sem1, sem2, sem3, sem4):
    c = lax.axis_index("c")
    s = lax.axis_index("s")
    wid = s * NC + c
    idxs = (idx0, idx1, idx2, idx3, idx4)
    segs = (seg0, seg1, seg2, seg3, seg4)
    sems = (sem0, sem1, sem2, sem3, sem4)

    def _zero_row(r, carry):
        zero16 = jnp.zeros((16,), jnp.float32)
        for q in range(ROW // 16):
            zbuf[r, pl.ds(q * 16, 16)] = zero16
        return carry

    lax.fori_loop(0, 16, _zero_row, 0)
    for d in range(ZROWS // 16):
        pltpu.sync_copy(zbuf, acc.at[pl.ds(s * ZROWS + d * 16, 16)])
    plsc.subcore_barrier()

    def _group(g, carry):
        for k in range(DEPTH):
            cidx = (g * DEPTH + k) * NW + wid

            @pl.when(cidx < N_CHUNKS)
            def _(k=k, cidx=cidx):
                off = cidx * CHUNK
                pltpu.sync_copy(ids_hbm.at[pl.ds(off, CHUNK)], idxs[k])
                pltpu.sync_copy(segs_hbm.at[pl.ds(off, CHUNK)], segs[k])
                pltpu.async_copy(table_hbm.at[idxs[k]], rows.at[k], sems[k])

        for k in range(DEPTH):
            cidx = (g * DEPTH + k) * NW + wid

            @pl.when(cidx < N_CHUNKS)
            def _(k=k):
                pltpu.make_async_copy(table_hbm.at[idxs[k]], rows.at[k],
                                      sems[k]).wait()
                pltpu.sync_copy(rows.at[k], acc.at[segs[k]], add=True)

        return carry

    lax.fori_loop(0, GROUPS, _group, 0)
    plsc.subcore_barrier()

    pltpu.sync_copy(acc.at[pl.ds(s * ZROWS, ZROWS)],
                    out_hbm.at[c, pl.ds(s * ZROWS, ZROWS)])


def _segment_bag(snp_ids, seg_ids, table):
    mesh = plsc.VectorSubcoreMesh(core_axis_name="c", subcore_axis_name="s")
    kern = functools.partial(
        pl.kernel,
        mesh=mesh,
        out_type=jax.ShapeDtypeStruct((NC, SEG_PAD, ROW), jnp.float32),
        scratch_types=(
            [pltpu.VMEM((CHUNK,), jnp.int32) for _ in range(10)]
            + [
                pltpu.VMEM((DEPTH, CHUNK, ROW), jnp.float32),
                pltpu.VMEM((16, ROW), jnp.float32),
                pltpu.VMEM_SHARED((SEG_PAD, ROW), jnp.float32),
            ]
            + [pltpu.SemaphoreType.DMA for _ in range(5)]
        ),
    )(_bag_body)
    return kern(snp_ids, seg_ids, table)


def _head_body(p2_ref, W1_ref, b1_ref, g1_ref, be1_ref,
               W2_ref, b2_ref, g2_ref, be2_ref, Wp_ref, bp_ref, out_ref):
    acc = (p2_ref[0] + p2_ref[1])[:N_GENES, :B]
    inv = 1.0 / (1.0 + EPS) ** 0.5
    x = lax.dot_general(acc, W1_ref[...], (((0,), (0,)), ((), ())),
                        preferred_element_type=jnp.float32)
    x = x + b1_ref[...]
    x = x * (inv * g1_ref[...]) + be1_ref[...]
    x = jnp.maximum(x, 0.0)
    x = jnp.dot(x, W2_ref[...], preferred_element_type=jnp.float32)
    x = x + b2_ref[...]
    x = x * (inv * g2_ref[...]) + be2_ref[...]
    x = jnp.maximum(x, 0.0)
    x = jnp.dot(x, Wp_ref[...], preferred_element_type=jnp.float32)
    out_ref[...] = x + bp_ref[...]


def _head(p2, W1, b1, g1, be1, W2, b2, g2, be2, Wp, bp):
    vecs = [v.reshape(1, -1) for v in (b1, g1, be1, b2, g2, be2, bp)]
    return pl.pallas_call(
        _head_body,
        out_shape=jax.ShapeDtypeStruct((B, 1), jnp.float32),
    )(p2, W1, vecs[0], vecs[1], vecs[2], W2, vecs[3], vecs[4], vecs[5],
      Wp, vecs[6])


def kernel(snp, snp_ids, seg_ids, filters, W1, b1, gamma1, beta1,
           W2, b2, gamma2, beta2, Wp, bp):
    table = _build_table(snp, filters)
    p2 = _segment_bag(snp_ids.astype(jnp.int32), seg_ids.astype(jnp.int32),
                      table)
    return _head(p2, W1, b1, gamma1, beta1, W2, b2, gamma2, beta2, Wp, bp)


# speedup vs baseline: 1.0356x; 1.0356x over previous
import functools

import jax
import jax.numpy as jnp
from jax import lax
from jax.experimental import pallas as pl
from jax.experimental.pallas import tpu as pltpu
from jax.experimental.pallas import tpu_sc as plsc

B = 32
N_SNPS = 50000
N_GENES = 5000
N_NODES = 160000
N_FILT = 8
EPS = 1e-5
ROW = 128

NC = 2
NS = 16
NW = NC * NS

CHUNK = 128
N_CHUNKS = N_NODES // CHUNK
CHUNKS_PER_TILE = -(-N_CHUNKS // NW)
DEPTH = 4
GROUPS = CHUNKS_PER_TILE // DEPTH
SEG_PAD = 5120
ZROWS = SEG_PAD // NS


def _prep_body(snp_ref, filt_ref, table_ref):
    fmean = jnp.sum(filt_ref[...], axis=1) * (1.0 / N_FILT)
    s_t = jnp.transpose(snp_ref[...])
    blk = s_t.shape[0]
    table_ref[...] = jnp.concatenate(
        [s_t * fmean[:, None], jnp.zeros((blk, ROW - B), jnp.float32)],
        axis=1)


_PREP_BLK = 2048
_PREP_GRID = -(-N_SNPS // _PREP_BLK)


def _build_table(snp, filters):
    return pl.pallas_call(
        _prep_body,
        grid=(_PREP_GRID,),
        in_specs=[
            pl.BlockSpec((B, _PREP_BLK), lambda i: (0, i)),
            pl.BlockSpec((_PREP_BLK, N_FILT), lambda i: (i, 0)),
        ],
        out_specs=pl.BlockSpec((_PREP_BLK, ROW), lambda i: (i, 0)),
        out_shape=jax.ShapeDtypeStruct((N_SNPS, ROW), jnp.float32),
    )(snp, filters)


def _bag_body(ids_hbm, segs_hbm, table_hbm, out_hbm,
              idx0, idx1, idx2, idx3, seg0, seg1, seg2, seg3,
              rows, zbuf, acc, sem0, sem1, sem2, sem3):
    c = lax.axis_index("c")
    s = lax.axis_index("s")
    wid = s * NC + c
    idxs = (idx0, idx1, idx2, idx3)
    segs = (seg0, seg1, seg2, seg3)
    sems = (sem0, sem1, sem2, sem3)

    def _zero_row(r, carry):
        zero16 = jnp.zeros((16,), jnp.float32)
        for q in range(ROW // 16):
            zbuf[r, pl.ds(q * 16, 16)] = zero16
        return carry

    lax.fori_loop(0, ZROWS // DEPTH, _zero_row, 0)
    for d in range(DEPTH):
        pltpu.sync_copy(zbuf,
                        acc.at[pl.ds(s * ZROWS + d * (ZROWS // DEPTH),
                                     ZROWS // DEPTH)])
    plsc.subcore_barrier()

    def _fire(k, cidx):
        off = cidx * CHUNK
        pltpu.sync_copy(ids_hbm.at[pl.ds(off, CHUNK)], idxs[k])
        pltpu.sync_copy(segs_hbm.at[pl.ds(off, CHUNK)], segs[k])
        pltpu.async_copy(table_hbm.at[idxs[k]], rows.at[k], sems[k])

    for k in range(DEPTH):
        _fire(k, k * NW + wid)

    def _group(g, carry):
        for k in range(DEPTH):
            cidx = (g * DEPTH + k) * NW + wid

            @pl.when(cidx < N_CHUNKS)
            def _(k=k, cidx=cidx):
                pltpu.make_async_copy(table_hbm.at[idxs[k]], rows.at[k],
                                      sems[k]).wait()
                pltpu.sync_copy(rows.at[k], acc.at[segs[k]], add=True)

            cnext = ((g + 1) * DEPTH + k) * NW + wid

            @pl.when(cnext < N_CHUNKS)
            def _(k=k, cnext=cnext):
                _fire(k, cnext)

        return carry

    lax.fori_loop(0, GROUPS, _group, 0)
    plsc.subcore_barrier()

    pltpu.sync_copy(acc.at[pl.ds(s * ZROWS, ZROWS)],
                    out_hbm.at[c, pl.ds(s * ZROWS, ZROWS)])


def _segment_bag(snp_ids, seg_ids, table):
    mesh = plsc.VectorSubcoreMesh(core_axis_name="c", subcore_axis_name="s")
    kern = functools.partial(
        pl.kernel,
        mesh=mesh,
        out_type=jax.ShapeDtypeStruct((NC, SEG_PAD, ROW), jnp.float32),
        scratch_types=(
            [pltpu.VMEM((CHUNK,), jnp.int32) for _ in range(8)]
            + [
                pltpu.VMEM((DEPTH, CHUNK, ROW), jnp.float32),
                pltpu.VMEM((ZROWS // DEPTH, ROW), jnp.float32),
                pltpu.VMEM_SHARED((SEG_PAD, ROW), jnp.float32),
            ]
            + [pltpu.SemaphoreType.DMA for _ in range(4)]
        ),
    )(_bag_body)
    return kern(snp_ids, seg_ids, table)


def _head_body(p2_ref, W1_ref, b1_ref, g1_ref, be1_ref,
               W2_ref, b2_ref, g2_ref, be2_ref, Wp_ref, bp_ref, out_ref):
    acc = (p2_ref[0] + p2_ref[1])[:N_GENES, :B]
    inv = 1.0 / (1.0 + EPS) ** 0.5
    x = lax.dot_general(acc, W1_ref[...], (((0,), (0,)), ((), ())),
                        preferred_element_type=jnp.float32)
    x = x + b1_ref[...]
    x = x * (inv * g1_ref[...]) + be1_ref[...]
    x = jnp.maximum(x, 0.0)
    x = jnp.dot(x, W2_ref[...], preferred_element_type=jnp.float32)
    x = x + b2_ref[...]
    x = x * (inv * g2_ref[...]) + be2_ref[...]
    x = jnp.maximum(x, 0.0)
    x = jnp.dot(x, Wp_ref[...], preferred_element_type=jnp.float32)
    out_ref[...] = x + bp_ref[...]


def _head(p2, W1, b1, g1, be1, W2, b2, g2, be2, Wp, bp):
    vecs = [v.reshape(1, -1) for v in (b1, g1, be1, b2, g2, be2, bp)]
    return pl.pallas_call(
        _head_body,
        out_shape=jax.ShapeDtypeStruct((B, 1), jnp.float32),
    )(p2, W1, vecs[0], vecs[1], vecs[2], W2, vecs[3], vecs[4], vecs[5],
      Wp, vecs[6])


def kernel(snp, snp_ids, seg_ids, filters, W1, b1, gamma1, beta1,
           W2, b2, gamma2, beta2, Wp, bp):
    table = _build_table(snp, filters)
    p2 = _segment_bag(snp_ids.astype(jnp.int32), seg_ids.astype(jnp.int32),
                      table)
    return _head(p2, W1, b1, gamma1, beta1, W2, b2, gamma2, beta2, Wp, bp)
